# baseline (speedup 1.0000x reference)
import functools

import jax
import jax.numpy as jnp
from jax import lax
from jax.experimental import pallas as pl
from jax.experimental.pallas import tpu as pltpu
from jax.experimental.pallas import tpu_sc as plsc

N_NODES = 10000
N_EDGES = 320000
D_IN = 128
D_HID = 128

NC = 2
NS = 16
NW = NC * NS
EPT = N_EDGES // NW
CH = 128
CPT = 80
EPADT = CPT * CH
TRASH = N_NODES
ACC_ROWS = N_NODES + 8
NBUF = 2
NIB = 4
RPT = 624
REM = N_NODES - NS * RPT
ZREM = ACC_ROWS - NS * RPT

_BLK = 2000
_GRID = N_NODES // _BLK


def _build_sc_sync(with_cnt):
    D = D_HID
    mesh = plsc.VectorSubcoreMesh(core_axis_name="c", subcore_axis_name="s",
                                  num_cores=NC, num_subcores=NS)
    out_type = [jax.ShapeDtypeStruct((2 * N_NODES, D), jnp.float32)]
    if with_cnt:
        out_type.append(jax.ShapeDtypeStruct((2 * N_NODES, 16), jnp.float32))
    scratch = [
        pltpu.VMEM((CPT, CH), jnp.int32),
        pltpu.VMEM((CPT, CH), jnp.int32),
        pltpu.VMEM((CH, D), jnp.float32),
    ]
    if with_cnt:
        scratch += [pltpu.VMEM((CH, 16), jnp.float32)]
    scratch += [pltpu.VMEM_SHARED((ACC_ROWS, D), jnp.float32)]
    if with_cnt:
        scratch += [pltpu.VMEM_SHARED((ACC_ROWS, 16), jnp.float32)]
    scratch += [pltpu.SemaphoreType.DMA]

    @functools.partial(
        pl.kernel, mesh=mesh,
        compiler_params=pltpu.CompilerParams(use_tc_tiling_on_sc=False),
        out_type=out_type,
        scratch_types=scratch,
    )
    def k(*refs):
        if with_cnt:
            (src2_hbm, dst2_hbm, y_hbm, zd_hbm, z16_hbm, ones_hbm,
             out_hbm, cnt_hbm, idxs_v, idxd_v, rows_v, ones_v,
             acc_sh, cacc_sh, sem) = refs
        else:
            (src2_hbm, dst2_hbm, y_hbm, zd_hbm,
             out_hbm, idxs_v, idxd_v, rows_v, acc_sh, sem) = refs

        c = lax.axis_index("c")
        s = lax.axis_index("s")
        wid = c * NS + s
        rbase = s * RPT

        pltpu.sync_copy(zd_hbm, acc_sh.at[pl.ds(rbase, RPT)])
        if with_cnt:
            pltpu.sync_copy(z16_hbm, cacc_sh.at[pl.ds(rbase, RPT)])
            pltpu.sync_copy(ones_hbm, ones_v)

        @pl.when(s == 0)
        def _():
            pltpu.sync_copy(zd_hbm.at[pl.ds(0, ZREM)],
                            acc_sh.at[pl.ds(NS * RPT, ZREM)])
            if with_cnt:
                pltpu.sync_copy(z16_hbm.at[pl.ds(0, ZREM)],
                                cacc_sh.at[pl.ds(NS * RPT, ZREM)])

        pltpu.sync_copy(src2_hbm.at[pl.ds(wid * CPT, CPT)], idxs_v)
        pltpu.sync_copy(dst2_hbm.at[pl.ds(wid * CPT, CPT)], idxd_v)
        plsc.subcore_barrier()

        def chunk(g, carry):
            pltpu.async_copy(y_hbm.at[idxs_v.at[g]], rows_v, sem).wait()
            pltpu.sync_copy(rows_v, acc_sh.at[idxd_v.at[g]], add=True)
            if with_cnt:
                pltpu.sync_copy(ones_v, cacc_sh.at[idxd_v.at[g]], add=True)
            return carry

        lax.fori_loop(0, CPT, chunk, 0)
        plsc.subcore_barrier()

        obase = c * N_NODES + rbase
        pltpu.sync_copy(acc_sh.at[pl.ds(rbase, RPT)],
                        out_hbm.at[pl.ds(obase, RPT)])
        if with_cnt:
            pltpu.sync_copy(cacc_sh.at[pl.ds(rbase, RPT)],
                            cnt_hbm.at[pl.ds(obase, RPT)])

        @pl.when(s == 0)
        def _():
            pltpu.sync_copy(acc_sh.at[pl.ds(NS * RPT, REM)],
                            out_hbm.at[pl.ds(c * N_NODES + NS * RPT, REM)])
            if with_cnt:
                pltpu.sync_copy(cacc_sh.at[pl.ds(NS * RPT, REM)],
                                cnt_hbm.at[pl.ds(c * N_NODES + NS * RPT, REM)])

    return k


def _build_sc(D, with_cnt, stage_y):
    mesh = plsc.VectorSubcoreMesh(core_axis_name="c", subcore_axis_name="s",
                                  num_cores=NC, num_subcores=NS)
    out_type = [jax.ShapeDtypeStruct((2 * N_NODES, D), jnp.float32)]
    if with_cnt:
        out_type.append(jax.ShapeDtypeStruct((2 * N_NODES, 16), jnp.float32))
    scratch = [pltpu.VMEM((CH,), jnp.int32) for _ in range(2 * NIB)]
    scratch += [pltpu.VMEM((CH, D), jnp.float32) for _ in range(NBUF)]
    scratch += [pltpu.VMEM_SHARED((ACC_ROWS, D), jnp.float32)]
    if stage_y:
        scratch += [pltpu.VMEM_SHARED((N_NODES, D), jnp.float32)]
    if with_cnt:
        scratch += [pltpu.VMEM((CH, 16), jnp.float32),
                    pltpu.VMEM_SHARED((ACC_ROWS, 16), jnp.float32)]
    n_sems = 2 * NBUF + NIB + (NBUF if with_cnt else 0)
    scratch += [pltpu.SemaphoreType.DMA] * n_sems

    @functools.partial(
        pl.kernel, mesh=mesh,
        compiler_params=pltpu.CompilerParams(use_tc_tiling_on_sc=False),
        out_type=out_type,
        scratch_types=scratch,
    )
    def k(*refs):
        n_in = 6 if with_cnt else 4
        (src2_hbm, dst2_hbm, y_hbm, zd_hbm) = refs[:4]
        if with_cnt:
            z16_hbm, ones_hbm = refs[4:6]
        pos = n_in
        out_hbm = refs[pos]; pos += 1
        if with_cnt:
            cnt_hbm = refs[pos]; pos += 1
        idxs_ring = list(refs[pos:pos + NIB]); pos += NIB
        idxd_ring = list(refs[pos:pos + NIB]); pos += NIB
        rows = list(refs[pos:pos + NBUF]); pos += NBUF
        acc_sh = refs[pos]; pos += 1
        if stage_y:
            y_sh = refs[pos]; pos += 1
        if with_cnt:
            ones_v = refs[pos]; cacc_sh = refs[pos + 1]; pos += 2
        gsems = list(refs[pos:pos + NBUF]); pos += NBUF
        ssems = list(refs[pos:pos + NBUF]); pos += NBUF
        isems = list(refs[pos:pos + NIB]); pos += NIB
        if with_cnt:
            csems = list(refs[pos:pos + NBUF]); pos += NBUF

        c = lax.axis_index("c")
        s = lax.axis_index("s")
        wid = c * NS + s
        rbase = s * RPT

        pltpu.sync_copy(zd_hbm, acc_sh.at[pl.ds(rbase, RPT)])
        if with_cnt:
            pltpu.sync_copy(z16_hbm, cacc_sh.at[pl.ds(rbase, RPT)])
            pltpu.sync_copy(ones_hbm, ones_v)
        if stage_y:
            pltpu.sync_copy(y_hbm.at[pl.ds(rbase, RPT)],
                            y_sh.at[pl.ds(rbase, RPT)])

        @pl.when(s == 0)
        def _():
            pltpu.sync_copy(zd_hbm.at[pl.ds(0, ZREM)],
                            acc_sh.at[pl.ds(NS * RPT, ZREM)])
            if with_cnt:
                pltpu.sync_copy(z16_hbm.at[pl.ds(0, ZREM)],
                                cacc_sh.at[pl.ds(NS * RPT, ZREM)])
            if stage_y:
                pltpu.sync_copy(y_hbm.at[pl.ds(NS * RPT, REM)],
                                y_sh.at[pl.ds(NS * RPT, REM)])

        plsc.subcore_barrier()

        gsrc = y_sh if stage_y else y_hbm
        ibase = wid * CPT

        def fire_idx(g, q):
            pltpu.async_copy(src2_hbm.at[ibase + g], idxs_ring[q], isems[q])
            pltpu.async_copy(dst2_hbm.at[ibase + g], idxd_ring[q], isems[q])

        def wait_idx(q):
            pltpu.make_async_copy(src2_hbm.at[ibase], idxs_ring[q],
                                  isems[q]).wait()
            pltpu.make_async_copy(dst2_hbm.at[ibase], idxd_ring[q],
                                  isems[q]).wait()

        def fire_gather(q, rb):
            pltpu.async_copy(gsrc.at[idxs_ring[q]], rows[rb], gsems[rb])

        def wait_gather(rb):
            pltpu.make_async_copy(gsrc.at[idxs_ring[0]], rows[rb],
                                  gsems[rb]).wait()

        def fire_scatter(q, rb):
            pltpu.async_copy(rows[rb], acc_sh.at[idxd_ring[q]], ssems[rb],
                             add=True)
            if with_cnt:
                pltpu.async_copy(ones_v, cacc_sh.at[idxd_ring[q]], csems[rb],
                                 add=True)

        def wait_scatter(rb):
            pltpu.make_async_copy(rows[rb], acc_sh.at[idxd_ring[0]],
                                  ssems[rb]).wait()
            if with_cnt:
                pltpu.make_async_copy(ones_v, cacc_sh.at[idxd_ring[0]],
                                      csems[rb]).wait()

        for q in range(NIB):
            fire_idx(q, q)
        wait_idx(0)
        fire_gather(0, 0)

        def outer(i, carry):
            for b in range(NIB):
                g = i * NIB + b
                rb = b % NBUF
                wait_gather(rb)
                fire_scatter(b, rb)
                gf = g + 1
                rbf = (b + 1) % NBUF
                qf = (b + 1) % NIB

                @pl.when(gf < CPT)
                def _():
                    @pl.when(gf >= NBUF)
                    def _():
                        wait_scatter(rbf)
                        gi = g + NIB - 1
                        qi = (b + NIB - 1) % NIB

                        @pl.when(gi < CPT)
                        def _():
                            fire_idx(gi, qi)
                    wait_idx(qf)
                    fire_gather(qf, rbf)
            return carry

        lax.fori_loop(0, CPT // NIB, outer, 0)
        for rb in range(NBUF):
            wait_scatter(rb)
        plsc.subcore_barrier()

        obase = c * N_NODES + rbase
        pltpu.sync_copy(acc_sh.at[pl.ds(rbase, RPT)],
                        out_hbm.at[pl.ds(obase, RPT)])
        if with_cnt:
            pltpu.sync_copy(cacc_sh.at[pl.ds(rbase, RPT)],
                            cnt_hbm.at[pl.ds(obase, RPT)])

        @pl.when(s == 0)
        def _():
            pltpu.sync_copy(acc_sh.at[pl.ds(NS * RPT, REM)],
                            out_hbm.at[pl.ds(c * N_NODES + NS * RPT, REM)])
            if with_cnt:
                pltpu.sync_copy(cacc_sh.at[pl.ds(NS * RPT, REM)],
                                cnt_hbm.at[pl.ds(c * N_NODES + NS * RPT, REM)])

    return k


def _mm_t(a, w):
    return lax.dot_general(a, w, (((1,), (1,)), ((), ())),
                           preferred_element_type=jnp.float32)


def _pre_body(x_ref, wl_ref, wr_ref, u_ref, r_ref):
    xb = x_ref[...]
    u_ref[...] = _mm_t(xb, wl_ref[...])
    r_ref[...] = _mm_t(xb, wr_ref[...])


def _mid1_body(sa_ref, sb_ref, ca_ref, cb_ref, r_ref, b_ref, wl_ref, wr_ref,
               u2_ref, r2_ref, inv_ref):
    cnt = ca_ref[...][:, 0:1] + cb_ref[...][:, 0:1]
    inv = 1.0 / jnp.maximum(cnt, 1.0)
    h = jnp.maximum((sa_ref[...] + sb_ref[...]) * inv + b_ref[...] + r_ref[...],
                    0.0)
    u2_ref[...] = _mm_t(h, wl_ref[...])
    r2_ref[...] = _mm_t(h, wr_ref[...])
    inv_ref[...] = jnp.broadcast_to(inv, (inv.shape[0], 16))


def _mid2_body(sa_ref, sb_ref, inv_ref, r_ref, b_ref, wl_ref, wr_ref,
               u3_ref, r3_ref):
    inv = inv_ref[...][:, 0:1]
    h = jnp.maximum((sa_ref[...] + sb_ref[...]) * inv + b_ref[...] + r_ref[...],
                    0.0)
    u3_ref[...] = _mm_t(h, wl_ref[...])
    r3_ref[...] = _mm_t(h, wr_ref[...])


def _final_body(sa_ref, sb_ref, inv_ref, r_ref, b_ref, o_ref):
    inv = inv_ref[...][:, 0:1]
    o_ref[...] = (sa_ref[...] + sb_ref[...]) * inv + b_ref[...] + r_ref[...]


def _row_blk(d):
    return pl.BlockSpec((_BLK, d), lambda i: (i, 0))


def _row_blk_hi(d):
    return pl.BlockSpec((_BLK, d), lambda i: (i + _GRID, 0))


def _full_blk(r, c):
    return pl.BlockSpec((r, c), lambda i: (0, 0))


def kernel(x, edge_index, Wl1, Wr1, b1, Wl2, Wr2, b2, Wl3, Wr3, b3):
    f32 = jnp.float32
    i32 = jnp.int32
    src = edge_index[0].astype(i32)
    dst = edge_index[1].astype(i32)

    src2 = (jnp.zeros((NW, EPADT), i32)
            .at[:, :EPT].set(src.reshape(NW, EPT))
            .reshape(NW * CPT, CH))
    dst2 = (jnp.full((NW, EPADT), TRASH, i32)
            .at[:, :EPT].set(dst.reshape(NW, EPT))
            .reshape(NW * CPT, CH))

    zeros128 = jnp.zeros((RPT, D_HID), f32)
    zeros16 = jnp.zeros((RPT, 16), f32)
    ones16 = jnp.ones((CH, 16), f32)
    Wl3p = jnp.zeros((16, D_HID), f32).at[:2].set(Wl3)
    Wr3p = jnp.zeros((16, D_HID), f32).at[:2].set(Wr3)
    b3p = jnp.zeros((1, 16), f32).at[0, :2].set(b3)
    b1r = b1.reshape(1, D_HID)
    b2r = b2.reshape(1, D_HID)

    u1, r1 = pl.pallas_call(
        _pre_body,
        grid=(_GRID,),
        in_specs=[_row_blk(D_IN), _full_blk(D_HID, D_IN), _full_blk(D_HID, D_IN)],
        out_specs=[_row_blk(D_HID), _row_blk(D_HID)],
        out_shape=[jax.ShapeDtypeStruct((N_NODES, D_HID), f32)] * 2,
    )(x, Wl1, Wr1)

    s1, cnt = jax.tree.leaves(_build_sc_sync(with_cnt=True)(
        src2, dst2, u1, zeros128, zeros16, ones16))

    u2, r2, inv16 = pl.pallas_call(
        _mid1_body,
        grid=(_GRID,),
        in_specs=[_row_blk(D_HID), _row_blk_hi(D_HID),
                  _row_blk(16), _row_blk_hi(16),
                  _row_blk(D_HID), _full_blk(1, D_HID),
                  _full_blk(D_HID, D_HID), _full_blk(D_HID, D_HID)],
        out_specs=[_row_blk(D_HID), _row_blk(D_HID), _row_blk(16)],
        out_shape=[jax.ShapeDtypeStruct((N_NODES, D_HID), f32),
                   jax.ShapeDtypeStruct((N_NODES, D_HID), f32),
                   jax.ShapeDtypeStruct((N_NODES, 16), f32)],
    )(s1, s1, cnt, cnt, r1, b1r, Wl2, Wr2)

    s2 = jax.tree.leaves(_build_sc_sync(with_cnt=False)(
        src2, dst2, u2, zeros128))[0]

    u3, r3 = pl.pallas_call(
        _mid2_body,
        grid=(_GRID,),
        in_specs=[_row_blk(D_HID), _row_blk_hi(D_HID), _row_blk(16),
                  _row_blk(D_HID), _full_blk(1, D_HID),
                  _full_blk(16, D_HID), _full_blk(16, D_HID)],
        out_specs=[_row_blk(16), _row_blk(16)],
        out_shape=[jax.ShapeDtypeStruct((N_NODES, 16), f32)] * 2,
    )(s2, s2, inv16, r2, b2r, Wl3p, Wr3p)

    s3 = jax.tree.leaves(_build_sc(16, with_cnt=False, stage_y=True)(
        src2, dst2, u3, zeros16))[0]

    out16 = pl.pallas_call(
        _final_body,
        grid=(_GRID,),
        in_specs=[_row_blk(16), _row_blk_hi(16), _row_blk(16), _row_blk(16),
                  _full_blk(1, 16)],
        out_specs=_row_blk(16),
        out_shape=jax.ShapeDtypeStruct((N_NODES, 16), f32),
    )(s3, s3, inv16, r3, b3p)

    return out16[:, :2]


# speedup vs baseline: 1.8825x; 1.8825x over previous
import functools

import jax
import jax.numpy as jnp
from jax import lax
from jax.experimental import pallas as pl
from jax.experimental.pallas import tpu as pltpu
from jax.experimental.pallas import tpu_sc as plsc

N_NODES = 10000
N_EDGES = 320000
D_IN = 128
D_HID = 128

NC = 2
NS = 16
NW = NC * NS
EPT = N_EDGES // NW
CH = 128
CPT = 80
EPADT = CPT * CH
TRASH = N_NODES
ACC_ROWS = N_NODES + 8
NBUF = 2
NIB = 4
RPT = 624
REM = N_NODES - NS * RPT
ZREM = ACC_ROWS - NS * RPT

_BLK = 2000
_GRID = N_NODES // _BLK


def _build_sc_sync(with_cnt, ch=80, cpt=125):
    D = D_HID
    mesh = plsc.VectorSubcoreMesh(core_axis_name="c", subcore_axis_name="s",
                                  num_cores=NC, num_subcores=NS)
    out_type = [jax.ShapeDtypeStruct((2 * N_NODES, D), jnp.float32)]
    if with_cnt:
        out_type.append(jax.ShapeDtypeStruct((2 * N_NODES, 16), jnp.float32))
    scratch = [
        pltpu.VMEM((cpt, ch), jnp.int32),
        pltpu.VMEM((cpt, ch), jnp.int32),
        pltpu.VMEM((ch, D), jnp.float32),
    ]
    if with_cnt:
        scratch += [pltpu.VMEM((ch, 16), jnp.float32)]
    scratch += [pltpu.VMEM_SHARED((ACC_ROWS, D), jnp.float32)]
    if with_cnt:
        scratch += [pltpu.VMEM_SHARED((ACC_ROWS, 16), jnp.float32)]
    scratch += [pltpu.SemaphoreType.DMA]

    @functools.partial(
        pl.kernel, mesh=mesh,
        compiler_params=pltpu.CompilerParams(use_tc_tiling_on_sc=False),
        out_type=out_type,
        scratch_types=scratch,
    )
    def k(*refs):
        if with_cnt:
            (src2_hbm, dst2_hbm, y_hbm, zd_hbm, z16_hbm, ones_hbm,
             out_hbm, cnt_hbm, idxs_v, idxd_v, rows_v, ones_v,
             acc_sh, cacc_sh, sem) = refs
        else:
            (src2_hbm, dst2_hbm, y_hbm, zd_hbm,
             out_hbm, idxs_v, idxd_v, rows_v, acc_sh, sem) = refs

        c = lax.axis_index("c")
        s = lax.axis_index("s")
        wid = c * NS + s
        rbase = s * RPT

        pltpu.sync_copy(zd_hbm, acc_sh.at[pl.ds(rbase, RPT)])
        if with_cnt:
            pltpu.sync_copy(z16_hbm, cacc_sh.at[pl.ds(rbase, RPT)])
            pltpu.sync_copy(ones_hbm, ones_v)

        @pl.when(s == 0)
        def _():
            pltpu.sync_copy(zd_hbm.at[pl.ds(0, ZREM)],
                            acc_sh.at[pl.ds(NS * RPT, ZREM)])
            if with_cnt:
                pltpu.sync_copy(z16_hbm.at[pl.ds(0, ZREM)],
                                cacc_sh.at[pl.ds(NS * RPT, ZREM)])

        pltpu.sync_copy(src2_hbm.at[pl.ds(wid * cpt, cpt)], idxs_v)
        pltpu.sync_copy(dst2_hbm.at[pl.ds(wid * cpt, cpt)], idxd_v)
        plsc.subcore_barrier()

        def chunk(g, carry):
            pltpu.async_copy(y_hbm.at[idxs_v.at[g]], rows_v, sem).wait()
            pltpu.sync_copy(rows_v, acc_sh.at[idxd_v.at[g]], add=True)
            if with_cnt:
                pltpu.sync_copy(ones_v, cacc_sh.at[idxd_v.at[g]], add=True)
            return carry

        lax.fori_loop(0, cpt, chunk, 0)
        plsc.subcore_barrier()

        obase = c * N_NODES + rbase
        pltpu.sync_copy(acc_sh.at[pl.ds(rbase, RPT)],
                        out_hbm.at[pl.ds(obase, RPT)])
        if with_cnt:
            pltpu.sync_copy(cacc_sh.at[pl.ds(rbase, RPT)],
                            cnt_hbm.at[pl.ds(obase, RPT)])

        @pl.when(s == 0)
        def _():
            pltpu.sync_copy(acc_sh.at[pl.ds(NS * RPT, REM)],
                            out_hbm.at[pl.ds(c * N_NODES + NS * RPT, REM)])
            if with_cnt:
                pltpu.sync_copy(cacc_sh.at[pl.ds(NS * RPT, REM)],
                                cnt_hbm.at[pl.ds(c * N_NODES + NS * RPT, REM)])

    return k


def _build_sc(D, with_cnt, stage_y):
    mesh = plsc.VectorSubcoreMesh(core_axis_name="c", subcore_axis_name="s",
                                  num_cores=NC, num_subcores=NS)
    out_type = [jax.ShapeDtypeStruct((2 * N_NODES, D), jnp.float32)]
    if with_cnt:
        out_type.append(jax.ShapeDtypeStruct((2 * N_NODES, 16), jnp.float32))
    scratch = [pltpu.VMEM((CH,), jnp.int32) for _ in range(2 * NIB)]
    scratch += [pltpu.VMEM((CH, D), jnp.float32) for _ in range(NBUF)]
    scratch += [pltpu.VMEM_SHARED((ACC_ROWS, D), jnp.float32)]
    if stage_y:
        scratch += [pltpu.VMEM_SHARED((N_NODES, D), jnp.float32)]
    if with_cnt:
        scratch += [pltpu.VMEM((CH, 16), jnp.float32),
                    pltpu.VMEM_SHARED((ACC_ROWS, 16), jnp.float32)]
    n_sems = 2 * NBUF + NIB + (NBUF if with_cnt else 0)
    scratch += [pltpu.SemaphoreType.DMA] * n_sems

    @functools.partial(
        pl.kernel, mesh=mesh,
        compiler_params=pltpu.CompilerParams(use_tc_tiling_on_sc=False),
        out_type=out_type,
        scratch_types=scratch,
    )
    def k(*refs):
        n_in = 6 if with_cnt else 4
        (src2_hbm, dst2_hbm, y_hbm, zd_hbm) = refs[:4]
        if with_cnt:
            z16_hbm, ones_hbm = refs[4:6]
        pos = n_in
        out_hbm = refs[pos]; pos += 1
        if with_cnt:
            cnt_hbm = refs[pos]; pos += 1
        idxs_ring = list(refs[pos:pos + NIB]); pos += NIB
        idxd_ring = list(refs[pos:pos + NIB]); pos += NIB
        rows = list(refs[pos:pos + NBUF]); pos += NBUF
        acc_sh = refs[pos]; pos += 1
        if stage_y:
            y_sh = refs[pos]; pos += 1
        if with_cnt:
            ones_v = refs[pos]; cacc_sh = refs[pos + 1]; pos += 2
        gsems = list(refs[pos:pos + NBUF]); pos += NBUF
        ssems = list(refs[pos:pos + NBUF]); pos += NBUF
        isems = list(refs[pos:pos + NIB]); pos += NIB
        if with_cnt:
            csems = list(refs[pos:pos + NBUF]); pos += NBUF

        c = lax.axis_index("c")
        s = lax.axis_index("s")
        wid = c * NS + s
        rbase = s * RPT

        pltpu.sync_copy(zd_hbm, acc_sh.at[pl.ds(rbase, RPT)])
        if with_cnt:
            pltpu.sync_copy(z16_hbm, cacc_sh.at[pl.ds(rbase, RPT)])
            pltpu.sync_copy(ones_hbm, ones_v)
        if stage_y:
            pltpu.sync_copy(y_hbm.at[pl.ds(rbase, RPT)],
                            y_sh.at[pl.ds(rbase, RPT)])

        @pl.when(s == 0)
        def _():
            pltpu.sync_copy(zd_hbm.at[pl.ds(0, ZREM)],
                            acc_sh.at[pl.ds(NS * RPT, ZREM)])
            if with_cnt:
                pltpu.sync_copy(z16_hbm.at[pl.ds(0, ZREM)],
                                cacc_sh.at[pl.ds(NS * RPT, ZREM)])
            if stage_y:
                pltpu.sync_copy(y_hbm.at[pl.ds(NS * RPT, REM)],
                                y_sh.at[pl.ds(NS * RPT, REM)])

        plsc.subcore_barrier()

        gsrc = y_sh if stage_y else y_hbm
        ibase = wid * CPT

        def fire_idx(g, q):
            pltpu.async_copy(src2_hbm.at[ibase + g], idxs_ring[q], isems[q])
            pltpu.async_copy(dst2_hbm.at[ibase + g], idxd_ring[q], isems[q])

        def wait_idx(q):
            pltpu.make_async_copy(src2_hbm.at[ibase], idxs_ring[q],
                                  isems[q]).wait()
            pltpu.make_async_copy(dst2_hbm.at[ibase], idxd_ring[q],
                                  isems[q]).wait()

        def fire_gather(q, rb):
            pltpu.async_copy(gsrc.at[idxs_ring[q]], rows[rb], gsems[rb])

        def wait_gather(rb):
            pltpu.make_async_copy(gsrc.at[idxs_ring[0]], rows[rb],
                                  gsems[rb]).wait()

        def fire_scatter(q, rb):
            pltpu.async_copy(rows[rb], acc_sh.at[idxd_ring[q]], ssems[rb],
                             add=True)
            if with_cnt:
                pltpu.async_copy(ones_v, cacc_sh.at[idxd_ring[q]], csems[rb],
                                 add=True)

        def wait_scatter(rb):
            pltpu.make_async_copy(rows[rb], acc_sh.at[idxd_ring[0]],
                                  ssems[rb]).wait()
            if with_cnt:
                pltpu.make_async_copy(ones_v, cacc_sh.at[idxd_ring[0]],
                                      csems[rb]).wait()

        for q in range(NIB):
            fire_idx(q, q)
        wait_idx(0)
        fire_gather(0, 0)

        def outer(i, carry):
            for b in range(NIB):
                g = i * NIB + b
                rb = b % NBUF
                wait_gather(rb)
                fire_scatter(b, rb)
                gf = g + 1
                rbf = (b + 1) % NBUF
                qf = (b + 1) % NIB

                @pl.when(gf < CPT)
                def _():
                    @pl.when(gf >= NBUF)
                    def _():
                        wait_scatter(rbf)
                        gi = g + NIB - 1
                        qi = (b + NIB - 1) % NIB

                        @pl.when(gi < CPT)
                        def _():
                            fire_idx(gi, qi)
                    wait_idx(qf)
                    fire_gather(qf, rbf)
            return carry

        lax.fori_loop(0, CPT // NIB, outer, 0)
        for rb in range(NBUF):
            wait_scatter(rb)
        plsc.subcore_barrier()

        obase = c * N_NODES + rbase
        pltpu.sync_copy(acc_sh.at[pl.ds(rbase, RPT)],
                        out_hbm.at[pl.ds(obase, RPT)])
        if with_cnt:
            pltpu.sync_copy(cacc_sh.at[pl.ds(rbase, RPT)],
                            cnt_hbm.at[pl.ds(obase, RPT)])

        @pl.when(s == 0)
        def _():
            pltpu.sync_copy(acc_sh.at[pl.ds(NS * RPT, REM)],
                            out_hbm.at[pl.ds(c * N_NODES + NS * RPT, REM)])
            if with_cnt:
                pltpu.sync_copy(cacc_sh.at[pl.ds(NS * RPT, REM)],
                                cnt_hbm.at[pl.ds(c * N_NODES + NS * RPT, REM)])

    return k


def _mm_t(a, w):
    return lax.dot_general(a, w, (((1,), (1,)), ((), ())),
                           preferred_element_type=jnp.float32)


def _pre_body(x_ref, wl_ref, wr_ref, u_ref, r_ref):
    xb = x_ref[...]
    u_ref[...] = _mm_t(xb, wl_ref[...])
    r_ref[...] = _mm_t(xb, wr_ref[...])


def _mid1_body(sa_ref, sb_ref, ca_ref, cb_ref, r_ref, b_ref, wl_ref, wr_ref,
               u2_ref, r2_ref, inv_ref):
    cnt = ca_ref[...][:, 0:1] + cb_ref[...][:, 0:1]
    inv = 1.0 / jnp.maximum(cnt, 1.0)
    h = jnp.maximum((sa_ref[...] + sb_ref[...]) * inv + b_ref[...] + r_ref[...],
                    0.0)
    u2_ref[...] = _mm_t(h, wl_ref[...])
    r2_ref[...] = _mm_t(h, wr_ref[...])
    inv_ref[...] = jnp.broadcast_to(inv, (inv.shape[0], 16))


def _mid2_body(sa_ref, sb_ref, inv_ref, r_ref, b_ref, wl_ref, wr_ref,
               u3_ref, r3_ref):
    inv = inv_ref[...][:, 0:1]
    h = jnp.maximum((sa_ref[...] + sb_ref[...]) * inv + b_ref[...] + r_ref[...],
                    0.0)
    u3_ref[...] = _mm_t(h, wl_ref[...])
    r3_ref[...] = _mm_t(h, wr_ref[...])


def _final_body(sa_ref, sb_ref, inv_ref, r_ref, b_ref, o_ref):
    inv = inv_ref[...][:, 0:1]
    o_ref[...] = (sa_ref[...] + sb_ref[...]) * inv + b_ref[...] + r_ref[...]


def _row_blk(d):
    return pl.BlockSpec((_BLK, d), lambda i: (i, 0))


def _row_blk_hi(d):
    return pl.BlockSpec((_BLK, d), lambda i: (i + _GRID, 0))


def _full_blk(r, c):
    return pl.BlockSpec((r, c), lambda i: (0, 0))


def kernel(x, edge_index, Wl1, Wr1, b1, Wl2, Wr2, b2, Wl3, Wr3, b3):
    f32 = jnp.float32
    i32 = jnp.int32
    src = edge_index[0].astype(i32)
    dst = edge_index[1].astype(i32)

    src2 = (jnp.zeros((NW, EPADT), i32)
            .at[:, :EPT].set(src.reshape(NW, EPT))
            .reshape(NW * CPT, CH))
    dst2 = (jnp.full((NW, EPADT), TRASH, i32)
            .at[:, :EPT].set(dst.reshape(NW, EPT))
            .reshape(NW * CPT, CH))
    src2s = src.reshape(NW * 125, 80)
    dst2s = dst.reshape(NW * 125, 80)

    zeros128 = jnp.zeros((RPT, D_HID), f32)
    zeros16 = jnp.zeros((RPT, 16), f32)
    ones16 = jnp.ones((80, 16), f32)
    Wl3p = jnp.zeros((16, D_HID), f32).at[:2].set(Wl3)
    Wr3p = jnp.zeros((16, D_HID), f32).at[:2].set(Wr3)
    b3p = jnp.zeros((1, 16), f32).at[0, :2].set(b3)
    b1r = b1.reshape(1, D_HID)
    b2r = b2.reshape(1, D_HID)

    u1, r1 = pl.pallas_call(
        _pre_body,
        grid=(_GRID,),
        in_specs=[_row_blk(D_IN), _full_blk(D_HID, D_IN), _full_blk(D_HID, D_IN)],
        out_specs=[_row_blk(D_HID), _row_blk(D_HID)],
        out_shape=[jax.ShapeDtypeStruct((N_NODES, D_HID), f32)] * 2,
    )(x, Wl1, Wr1)

    s1, cnt = jax.tree.leaves(_build_sc_sync(with_cnt=True)(
        src2s, dst2s, u1, zeros128, zeros16, ones16))

    u2, r2, inv16 = pl.pallas_call(
        _mid1_body,
        grid=(_GRID,),
        in_specs=[_row_blk(D_HID), _row_blk_hi(D_HID),
                  _row_blk(16), _row_blk_hi(16),
                  _row_blk(D_HID), _full_blk(1, D_HID),
                  _full_blk(D_HID, D_HID), _full_blk(D_HID, D_HID)],
        out_specs=[_row_blk(D_HID), _row_blk(D_HID), _row_blk(16)],
        out_shape=[jax.ShapeDtypeStruct((N_NODES, D_HID), f32),
                   jax.ShapeDtypeStruct((N_NODES, D_HID), f32),
                   jax.ShapeDtypeStruct((N_NODES, 16), f32)],
    )(s1, s1, cnt, cnt, r1, b1r, Wl2, Wr2)

    s2 = jax.tree.leaves(_build_sc_sync(with_cnt=False)(
        src2s, dst2s, u2, zeros128))[0]

    u3, r3 = pl.pallas_call(
        _mid2_body,
        grid=(_GRID,),
        in_specs=[_row_blk(D_HID), _row_blk_hi(D_HID), _row_blk(16),
                  _row_blk(D_HID), _full_blk(1, D_HID),
                  _full_blk(16, D_HID), _full_blk(16, D_HID)],
        out_specs=[_row_blk(16), _row_blk(16)],
        out_shape=[jax.ShapeDtypeStruct((N_NODES, 16), f32)] * 2,
    )(s2, s2, inv16, r2, b2r, Wl3p, Wr3p)

    s3 = jax.tree.leaves(_build_sc(16, with_cnt=False, stage_y=True)(
        src2, dst2, u3, zeros16))[0]

    out16 = pl.pallas_call(
        _final_body,
        grid=(_GRID,),
        in_specs=[_row_blk(16), _row_blk_hi(16), _row_blk(16), _row_blk(16),
                  _full_blk(1, 16)],
        out_specs=_row_blk(16),
        out_shape=jax.ShapeDtypeStruct((N_NODES, 16), f32),
    )(s3, s3, inv16, r3, b3p)

    return out16[:, :2]
